# baseline (device time: 38699 ns/iter reference)
import math

import jax
import jax.numpy as jnp
from jax import lax
from jax.experimental import pallas as pl
from jax.experimental.pallas import tpu as pltpu

N_DEV = 16
B = 2
SQ = 128
D = 512
HQ = 4
DH = 64
DQK = HQ * DH
ROWS = B * SQ
R_HOPS = 8
L_HOPS = 7


def kernel(x, Wq, Wk, Wv, Wo):
    def body(x_ref, wq_ref, wk_ref, wv_ref, wo_ref, out_ref,
             kv_ref, send_r, recv_r, send_l, recv_l):
        my = lax.axis_index("i")

        def ring_pos(m):
            z, c = m // 4, lax.rem(m, 4)
            return c * 4 + jnp.where(lax.rem(c, 2) == 0, z, 3 - z)

        def ring_to_logical(r):
            c, w = r // 4, lax.rem(r, 4)
            z = jnp.where(lax.rem(c, 2) == 0, w, 3 - w)
            return 4 * z + c

        my_r = ring_pos(my)
        right = ring_to_logical(lax.rem(my_r + 1, N_DEV))
        left = ring_to_logical(lax.rem(my_r + N_DEV - 1, N_DEV))

        x2 = x_ref[...].reshape(ROWS, D).astype(jnp.bfloat16)
        q = jnp.dot(x2, wq_ref[...].astype(jnp.bfloat16),
                    preferred_element_type=jnp.float32)
        k = jnp.dot(x2, wk_ref[...].astype(jnp.bfloat16),
                    preferred_element_type=jnp.float32)
        v = jnp.dot(x2, wv_ref[...].astype(jnp.bfloat16),
                    preferred_element_type=jnp.float32)

        row = lax.broadcasted_iota(jnp.int32, (ROWS, DQK), 0)
        col = lax.broadcasted_iota(jnp.int32, (ROWS, DQK), 1)
        pos = (lax.rem(row, SQ) + my * SQ).astype(jnp.float32)
        expo = (((lax.rem(col, DH) // 2) * 2).astype(jnp.float32)) / DH
        inv = jnp.exp(-expo * math.log(10000.0))
        angle = pos * inv
        cosv = jnp.cos(angle)
        sinv = jnp.sin(angle)

        jj = lax.broadcasted_iota(jnp.int32, (DQK, DQK), 0)
        cc = lax.broadcasted_iota(jnp.int32, (DQK, DQK), 1)
        rot = jnp.where((lax.rem(cc, 2) == 0) & (jj == cc + 1), -1.0,
                        jnp.where((lax.rem(cc, 2) == 1) & (jj == cc - 1),
                                  1.0, 0.0)).astype(jnp.bfloat16)

        q = q * cosv + jnp.dot(q.astype(jnp.bfloat16), rot,
                               preferred_element_type=jnp.float32) * sinv
        k = k * cosv + jnp.dot(k.astype(jnp.bfloat16), rot,
                               preferred_element_type=jnp.float32) * sinv

        kv_ref[0] = jnp.concatenate([k, v], axis=1).astype(jnp.bfloat16)

        barrier = pltpu.get_barrier_semaphore()
        for nbr in (left, right):
            pl.semaphore_signal(barrier, inc=1, device_id=(nbr,),
                                device_id_type=pl.DeviceIdType.MESH)
        pl.semaphore_wait(barrier, 2)

        lsum = [jnp.zeros((SQ, 1), jnp.float32) for _ in range(B * HQ)]
        acc = [jnp.zeros((SQ, DH), jnp.float32) for _ in range(B * HQ)]

        q_bf = (q * 0.125).astype(jnp.bfloat16)

        def process_b(slot, b):
            r0 = b * SQ
            for hh in range(HQ):
                c0 = hh * DH
                i = b * HQ + hh
                qbh = q_bf[r0:r0 + SQ, c0:c0 + DH]
                kc = kv_ref[slot, r0:r0 + SQ, c0:c0 + DH]
                vc = kv_ref[slot, r0:r0 + SQ, DQK + c0:DQK + c0 + DH]
                s = lax.dot_general(
                    qbh, kc, (((1,), (1,)), ((), ())),
                    preferred_element_type=jnp.float32)
                p = jnp.exp(s)
                lsum[i] = lsum[i] + jnp.sum(p, axis=1, keepdims=True)
                acc[i] = acc[i] + jnp.dot(
                    p.astype(jnp.bfloat16), vc,
                    preferred_element_type=jnp.float32)

        QROWS = ROWS // 4

        def start(src_slot, dst_slot, sems_s, sems_r, h, qtr, dst_dev):
            rows = slice(qtr * QROWS, (qtr + 1) * QROWS)
            rdma = pltpu.make_async_remote_copy(
                src_ref=kv_ref.at[src_slot, rows],
                dst_ref=kv_ref.at[dst_slot, rows],
                send_sem=sems_s.at[h, qtr],
                recv_sem=sems_r.at[h, qtr],
                device_id=(dst_dev,),
                device_id_type=pl.DeviceIdType.MESH,
            )
            rdma.start()
            return rdma

        dR = [[None] * 4 for _ in range(R_HOPS)]
        dL = [[None] * 4 for _ in range(L_HOPS)]

        for qtr in range(4):
            dR[0][qtr] = start(0, 1, send_r, recv_r, 0, qtr, right)
            dL[0][qtr] = start(0, 15, send_l, recv_l, 0, qtr, left)
        process_b(0, 0)
        process_b(0, 1)

        for h in range(1, R_HOPS + 1):
            for qtr in range(4):
                dR[h - 1][qtr].wait_recv()
                if h < R_HOPS:
                    dR[h][qtr] = start(
                        h, h + 1, send_r, recv_r, h, qtr, right)
                if h <= L_HOPS:
                    dL[h - 1][qtr].wait_recv()
                    if h < L_HOPS:
                        dL[h][qtr] = start(
                            16 - h, 15 - h, send_l, recv_l, h, qtr, left)
            for half in (0, 1):
                process_b(h, half)
                if h <= L_HOPS:
                    process_b(16 - h, half)

        ctx = jnp.concatenate(
            [jnp.concatenate([acc[b * HQ + hh] / lsum[b * HQ + hh]
                              for hh in range(HQ)], axis=1)
             for b in range(B)], axis=0)
        out2 = jnp.dot(ctx.astype(jnp.bfloat16),
                       wo_ref[...].astype(jnp.bfloat16),
                       preferred_element_type=jnp.float32)
        out_ref[...] = out2.reshape(B, SQ, D)

        for ds in dR + dL:
            for r in ds:
                r.wait_send()

    return pl.pallas_call(
        body,
        out_shape=jax.ShapeDtypeStruct((B, SQ, D), jnp.float32),
        in_specs=[pl.BlockSpec(memory_space=pltpu.VMEM)] * 5,
        out_specs=pl.BlockSpec(memory_space=pltpu.VMEM),
        scratch_shapes=[
            pltpu.VMEM((N_DEV, ROWS, 2 * DQK), jnp.bfloat16),
            pltpu.SemaphoreType.DMA((R_HOPS, 4)),
            pltpu.SemaphoreType.DMA((R_HOPS, 4)),
            pltpu.SemaphoreType.DMA((L_HOPS, 4)),
            pltpu.SemaphoreType.DMA((L_HOPS, 4)),
        ],
        compiler_params=pltpu.CompilerParams(collective_id=0),
    )(x, Wq, Wk, Wv, Wo)


# device time: 38150 ns/iter; 1.0144x vs baseline; 1.0144x over previous
import math

import jax
import jax.numpy as jnp
from jax import lax
from jax.experimental import pallas as pl
from jax.experimental.pallas import tpu as pltpu

N_DEV = 16
B = 2
SQ = 128
D = 512
HQ = 4
DH = 64
DQK = HQ * DH
ROWS = B * SQ
R_HOPS = 8
L_HOPS = 7


def kernel(x, Wq, Wk, Wv, Wo):
    def body(x_ref, wq_ref, wk_ref, wv_ref, wo_ref, out_ref,
             kv_ref, send_r, recv_r, send_l, recv_l):
        my = lax.axis_index("i")

        def ring_pos(m):
            z, c = m // 4, lax.rem(m, 4)
            return c * 4 + jnp.where(lax.rem(c, 2) == 0, z, 3 - z)

        def ring_to_logical(r):
            c, w = r // 4, lax.rem(r, 4)
            z = jnp.where(lax.rem(c, 2) == 0, w, 3 - w)
            return 4 * z + c

        my_r = ring_pos(my)
        right = ring_to_logical(lax.rem(my_r + 1, N_DEV))
        left = ring_to_logical(lax.rem(my_r + N_DEV - 1, N_DEV))

        x2 = x_ref[...].reshape(ROWS, D).astype(jnp.bfloat16)
        q = jnp.dot(x2, wq_ref[...].astype(jnp.bfloat16),
                    preferred_element_type=jnp.float32)
        k = jnp.dot(x2, wk_ref[...].astype(jnp.bfloat16),
                    preferred_element_type=jnp.float32)
        v = jnp.dot(x2, wv_ref[...].astype(jnp.bfloat16),
                    preferred_element_type=jnp.float32)

        row = lax.broadcasted_iota(jnp.int32, (ROWS, DQK), 0)
        col = lax.broadcasted_iota(jnp.int32, (ROWS, DQK), 1)
        pos = (lax.rem(row, SQ) + my * SQ).astype(jnp.float32)
        expo = (((lax.rem(col, DH) // 2) * 2).astype(jnp.float32)) / DH
        inv = jnp.exp(-expo * math.log(10000.0))
        angle = pos * inv
        cosv = jnp.cos(angle)
        sinv = jnp.sin(angle)

        jj = lax.broadcasted_iota(jnp.int32, (DQK, DQK), 0)
        cc = lax.broadcasted_iota(jnp.int32, (DQK, DQK), 1)
        rot = jnp.where((lax.rem(cc, 2) == 0) & (jj == cc + 1), -1.0,
                        jnp.where((lax.rem(cc, 2) == 1) & (jj == cc - 1),
                                  1.0, 0.0)).astype(jnp.bfloat16)

        q = q * cosv + jnp.dot(q.astype(jnp.bfloat16), rot,
                               preferred_element_type=jnp.float32) * sinv
        k = k * cosv + jnp.dot(k.astype(jnp.bfloat16), rot,
                               preferred_element_type=jnp.float32) * sinv

        kv_ref[0] = jnp.concatenate([k, v], axis=1).astype(jnp.bfloat16)

        barrier = pltpu.get_barrier_semaphore()
        for nbr in (left, right):
            pl.semaphore_signal(barrier, inc=1, device_id=(nbr,),
                                device_id_type=pl.DeviceIdType.MESH)
        pl.semaphore_wait(barrier, 2)

        lsum = [jnp.zeros((SQ, 1), jnp.float32) for _ in range(B * HQ)]
        acc = [jnp.zeros((SQ, DH), jnp.float32) for _ in range(B * HQ)]

        q_bf = (q * 0.125).astype(jnp.bfloat16)

        def process_b(slots, b):
            r0 = b * SQ
            for hh in range(HQ):
                c0 = hh * DH
                i = b * HQ + hh
                qbh = q_bf[r0:r0 + SQ, c0:c0 + DH]
                kc = jnp.concatenate(
                    [kv_ref[s, r0:r0 + SQ, c0:c0 + DH] for s in slots],
                    axis=0)
                vc = jnp.concatenate(
                    [kv_ref[s, r0:r0 + SQ, DQK + c0:DQK + c0 + DH]
                     for s in slots], axis=0)
                s = lax.dot_general(
                    qbh, kc, (((1,), (1,)), ((), ())),
                    preferred_element_type=jnp.float32)
                p = jnp.exp(s)
                lsum[i] = lsum[i] + jnp.sum(p, axis=1, keepdims=True)
                acc[i] = acc[i] + jnp.dot(
                    p.astype(jnp.bfloat16), vc,
                    preferred_element_type=jnp.float32)

        QROWS = ROWS // 4

        def start(src_slot, dst_slot, sems_s, sems_r, h, qtr, dst_dev):
            rows = slice(qtr * QROWS, (qtr + 1) * QROWS)
            rdma = pltpu.make_async_remote_copy(
                src_ref=kv_ref.at[src_slot, rows],
                dst_ref=kv_ref.at[dst_slot, rows],
                send_sem=sems_s.at[h, qtr],
                recv_sem=sems_r.at[h, qtr],
                device_id=(dst_dev,),
                device_id_type=pl.DeviceIdType.MESH,
            )
            rdma.start()
            return rdma

        dR = [[None] * 4 for _ in range(R_HOPS)]
        dL = [[None] * 4 for _ in range(L_HOPS)]

        for qtr in range(4):
            dR[0][qtr] = start(0, 1, send_r, recv_r, 0, qtr, right)
            dL[0][qtr] = start(0, 15, send_l, recv_l, 0, qtr, left)
        process_b((0,), 0)
        process_b((0,), 1)

        for h in range(1, R_HOPS + 1):
            for qtr in range(4):
                dR[h - 1][qtr].wait_recv()
                if h < R_HOPS:
                    dR[h][qtr] = start(
                        h, h + 1, send_r, recv_r, h, qtr, right)
                if h <= L_HOPS:
                    dL[h - 1][qtr].wait_recv()
                    if h < L_HOPS:
                        dL[h][qtr] = start(
                            16 - h, 15 - h, send_l, recv_l, h, qtr, left)
            slots = (h, 16 - h) if h <= L_HOPS else (h,)
            process_b(slots, 0)
            process_b(slots, 1)

        ctx = jnp.concatenate(
            [jnp.concatenate([acc[b * HQ + hh] / lsum[b * HQ + hh]
                              for hh in range(HQ)], axis=1)
             for b in range(B)], axis=0)
        out2 = jnp.dot(ctx.astype(jnp.bfloat16),
                       wo_ref[...].astype(jnp.bfloat16),
                       preferred_element_type=jnp.float32)
        out_ref[...] = out2.reshape(B, SQ, D)

        for ds in dR + dL:
            for r in ds:
                r.wait_send()

    return pl.pallas_call(
        body,
        out_shape=jax.ShapeDtypeStruct((B, SQ, D), jnp.float32),
        in_specs=[pl.BlockSpec(memory_space=pltpu.VMEM)] * 5,
        out_specs=pl.BlockSpec(memory_space=pltpu.VMEM),
        scratch_shapes=[
            pltpu.VMEM((N_DEV, ROWS, 2 * DQK), jnp.bfloat16),
            pltpu.SemaphoreType.DMA((R_HOPS, 4)),
            pltpu.SemaphoreType.DMA((R_HOPS, 4)),
            pltpu.SemaphoreType.DMA((L_HOPS, 4)),
            pltpu.SemaphoreType.DMA((L_HOPS, 4)),
        ],
        compiler_params=pltpu.CompilerParams(collective_id=0),
    )(x, Wq, Wk, Wv, Wo)
